# baseline (speedup 1.0000x reference)
import functools

import jax
import jax.numpy as jnp
from jax.experimental import pallas as pl
from jax.experimental.pallas import tpu as pltpu

F32 = jnp.float32
BF16 = jnp.bfloat16
F = 256
EPS = 1e-5


def _prologue_body(x_ref, g_ref, b_ref, w_ref, bi_ref, w1_ref, o_ref):
    x = x_ref[...]
    m = jnp.mean(x, axis=0, keepdims=True)
    v = jnp.mean((x - m) ** 2, axis=0, keepdims=True)
    xn = (x - m) / jnp.sqrt(v + EPS) * g_ref[...] + b_ref[...]
    h = jnp.dot(xn.astype(BF16), w_ref[...], preferred_element_type=F32)
    t = jnp.tanh(h + bi_ref[...]).astype(BF16)
    o_ref[...] = jnp.dot(t, w1_ref[...], preferred_element_type=F32).astype(BF16)


def _cast_spmm_body(a_ref, h_ref, abf_ref, o_ref):
    abf = a_ref[...].astype(BF16)
    abf_ref[...] = abf
    o_ref[...] = jnp.dot(abf, h_ref[...],
                         preferred_element_type=F32).astype(BF16)


def _spmm_plain_body(a_ref, h_ref, o_ref):
    o_ref[...] = jnp.dot(a_ref[...], h_ref[...],
                         preferred_element_type=F32).astype(BF16)


def _masked_layernorm(y, dval, g, b):
    m = jnp.sum(y, axis=1, keepdims=True) / dval
    lane = jax.lax.broadcasted_iota(jnp.int32, y.shape, 1)
    d = jnp.where(lane < dval, y - m, 0.0)
    v = jnp.sum(d * d, axis=1, keepdims=True) / dval
    return d / jnp.sqrt(v + EPS) * g + b


def _spmm_ln_body(a_ref, h_ref, w_ref, b_ref, g_ref, bb_ref, o_ref, *, dval):
    z = jnp.dot(a_ref[...], h_ref[...], preferred_element_type=F32)
    y = jnp.dot(z.astype(BF16), w_ref[...], preferred_element_type=F32) + b_ref[...]
    t = jnp.tanh(_masked_layernorm(y, dval, g_ref[...], bb_ref[...]))
    o_ref[...] = t.astype(BF16)


def _spmm_final_body(a_ref, h_ref, w_ref, b_ref, g_ref, bb_ref, wo_ref, bo_ref,
                     o_ref, *, dval):
    z = jnp.dot(a_ref[...], h_ref[...], preferred_element_type=F32)
    y = jnp.dot(z.astype(BF16), w_ref[...], preferred_element_type=F32) + b_ref[...]
    t = jnp.tanh(_masked_layernorm(y, dval, g_ref[...], bb_ref[...]))
    o_ref[...] = (jnp.dot(t.astype(BF16), wo_ref[...], preferred_element_type=F32)
                  + bo_ref[...])


def _spmm(a_bf, h, body, extras, out_w, out_dtype, r_blk, interpret=False):
    n_rows, n_cols = a_bf.shape
    nblk = -(-n_rows // r_blk)
    f_in = h.shape[1]
    row_spec = pl.BlockSpec((r_blk, n_cols), lambda i: (i, 0))
    whole = lambda shape: pl.BlockSpec(shape, lambda i: (0, 0))
    in_specs = [row_spec, whole((n_cols, f_in))] + [whole(e.shape) for e in extras]
    out_spec = pl.BlockSpec((r_blk, out_w), lambda i: (i, 0))
    return pl.pallas_call(
        body,
        grid=(nblk,),
        in_specs=in_specs,
        out_specs=out_spec,
        out_shape=jax.ShapeDtypeStruct((nblk * r_blk, out_w), out_dtype),
        compiler_params=pltpu.CompilerParams(
            dimension_semantics=("arbitrary",)),
        interpret=interpret,
    )(a_bf, h, *extras)


def _cast_spmm(adj, h, r_blk, interpret=False):
    n_rows, n_cols = adj.shape
    f_in = h.shape[1]
    row_spec = pl.BlockSpec((r_blk, n_cols), lambda i: (i, 0))
    return pl.pallas_call(
        _cast_spmm_body,
        grid=(n_rows // r_blk,),
        in_specs=[row_spec, pl.BlockSpec((n_cols, f_in), lambda i: (0, 0))],
        out_specs=[row_spec, pl.BlockSpec((r_blk, f_in), lambda i: (i, 0))],
        out_shape=[jax.ShapeDtypeStruct((n_rows, n_cols), BF16),
                   jax.ShapeDtypeStruct((n_rows, f_in), BF16)],
        compiler_params=pltpu.CompilerParams(
            dimension_semantics=("arbitrary",)),
        interpret=interpret,
    )(adj, h)


def _mega_body(a_ref, h1_ref, b1_ref, g1_ref, bb1_ref,
               w2_ref, b2_ref, g2_ref, bb2_ref, wo_ref, bo_ref,
               o_ref, sa, sb, *, r, n, d1, d2, f2):
    p = pl.program_id(0)
    i = pl.program_id(1)
    row0 = i * r

    @pl.when(p == 0)
    def _():
        z = jnp.dot(a_ref[...], h1_ref[...], preferred_element_type=F32)
        sa[pl.ds(row0, r), :] = z.astype(BF16)

    @pl.when(p == 1)
    def _():
        z = jnp.dot(a_ref[...], sa[:n, :], preferred_element_type=F32)
        sb[pl.ds(row0, r), :] = z.astype(BF16)

    @pl.when(p == 2)
    def _():
        z = jnp.dot(a_ref[...], sb[:n, :], preferred_element_type=F32)
        y = z + b1_ref[...]
        t = jnp.tanh(_masked_layernorm(y, d1, g1_ref[...], bb1_ref[...]))
        g = jnp.dot(t.astype(BF16), w2_ref[...], preferred_element_type=F32)
        sa[pl.ds(row0, r), :f2] = g.astype(BF16)

    @pl.when(p == 3)
    def _():
        z = jnp.dot(a_ref[...], sa[:n, :f2], preferred_element_type=F32)
        sb[pl.ds(row0, r), :f2] = z.astype(BF16)

    @pl.when(p == 4)
    def _():
        z = jnp.dot(a_ref[...], sb[:n, :f2], preferred_element_type=F32)
        sa[pl.ds(row0, r), :f2] = z.astype(BF16)

    @pl.when(p == 5)
    def _():
        z = jnp.dot(a_ref[...], sa[:n, :f2], preferred_element_type=F32)
        sb[pl.ds(row0, r), :f2] = z.astype(BF16)

    @pl.when(p == 6)
    def _():
        z = jnp.dot(a_ref[...], sb[:n, :f2], preferred_element_type=F32)
        y = z + b2_ref[...]
        t = jnp.tanh(_masked_layernorm(y, d2, g2_ref[...], bb2_ref[...]))
        o_ref[...] = (jnp.dot(t.astype(BF16), wo_ref[...],
                              preferred_element_type=F32) + bo_ref[...])


def _mega(a_bf, h1, extras, d1, d2, f2, d_out, r_blk, interpret=False):
    n, n_cols = a_bf.shape
    nblk = -(-n // r_blk)
    rows_pad = nblk * r_blk
    f_in = h1.shape[1]
    whole = lambda shape: pl.BlockSpec(shape, lambda p, i: (0, 0))
    body = functools.partial(_mega_body, r=r_blk, n=n_cols, d1=d1, d2=d2, f2=f2)
    return pl.pallas_call(
        body,
        grid=(7, nblk),
        in_specs=[pl.BlockSpec((r_blk, n_cols), lambda p, i: (i, 0)),
                  whole((n_cols, f_in))] + [whole(e.shape) for e in extras],
        out_specs=pl.BlockSpec((r_blk, d_out),
                               lambda p, i: (jnp.where(p == 6, i, 0), 0)),
        out_shape=jax.ShapeDtypeStruct((rows_pad, d_out), F32),
        scratch_shapes=[pltpu.VMEM((rows_pad, f_in), BF16),
                        pltpu.VMEM((rows_pad, f_in), BF16)],
        compiler_params=pltpu.CompilerParams(
            dimension_semantics=("arbitrary", "arbitrary"),
            vmem_limit_bytes=64 * 1024 * 1024),
        interpret=interpret,
    )(a_bf, h1, *extras)


def _pad2(a, rows, cols):
    return jnp.pad(a, ((0, rows - a.shape[0]), (0, cols - a.shape[1])))


def _row(v, width=F):
    return jnp.pad(v, (0, width - v.shape[0])).reshape(1, width)


def _pipeline(x, adj, bn_g, bn_b, Wi, bi, W1, b1, ln1_g, ln1_b, W2, b2,
              ln2_g, ln2_b, Wo, bo, r_blk, r_big, interpret=False):
    d_out = Wo.shape[0]
    f2 = 128

    wi_t = _pad2(Wi.T.astype(BF16), Wi.shape[1], F)
    w1_t = _pad2(W1.T.astype(BF16), F, f2)
    w2_t = _pad2(W2.T.astype(BF16), f2, f2)
    wo_t = _pad2(Wo.T.astype(BF16), f2, d_out)

    h = pl.pallas_call(
        _prologue_body,
        out_shape=jax.ShapeDtypeStruct((x.shape[0], f2), BF16),
        interpret=interpret,
    )(x, bn_g.reshape(1, -1), bn_b.reshape(1, -1), wi_t, _row(bi), w1_t)

    a_bf, h = _cast_spmm(adj, h, r_blk, interpret)
    out = _mega(a_bf, h,
                [_row(b1, f2), _row(ln1_g, f2), _row(ln1_b, f2),
                 w2_t, _row(b2, f2), _row(ln2_g, f2), _row(ln2_b, f2),
                 wo_t, _row(bo, d_out)],
                W1.shape[0], W2.shape[0], f2, d_out, r_big, interpret)
    return out[:adj.shape[0]]


def kernel(x, adj, bn_g, bn_b, Wi, bi, W1, b1, ln1_g, ln1_b, W2, b2,
           ln2_g, ln2_b, Wo, bo):
    return _pipeline(x, adj, bn_g, bn_b, Wi, bi, W1, b1, ln1_g, ln1_b,
                     W2, b2, ln2_g, ln2_b, Wo, bo, r_blk=400, r_big=1280)


# speedup vs baseline: 1.0297x; 1.0297x over previous
import functools

import jax
import jax.numpy as jnp
from jax.experimental import pallas as pl
from jax.experimental.pallas import tpu as pltpu

F32 = jnp.float32
BF16 = jnp.bfloat16
F = 256
EPS = 1e-5


def _prologue_body(x_ref, g_ref, b_ref, w_ref, bi_ref, w1_ref, o_ref):
    x = x_ref[...]
    m = jnp.mean(x, axis=0, keepdims=True)
    v = jnp.mean((x - m) ** 2, axis=0, keepdims=True)
    xn = (x - m) / jnp.sqrt(v + EPS) * g_ref[...] + b_ref[...]
    h = jnp.dot(xn.astype(BF16), w_ref[...], preferred_element_type=F32)
    t = jnp.tanh(h + bi_ref[...]).astype(BF16)
    o_ref[...] = jnp.dot(t, w1_ref[...], preferred_element_type=F32).astype(BF16)


def _cast_spmm_body(a_ref, h_ref, abf_ref, o_ref):
    abf = a_ref[...].astype(BF16)
    abf_ref[...] = abf
    o_ref[...] = jnp.dot(abf, h_ref[...],
                         preferred_element_type=F32).astype(BF16)


def _spmm_plain_body(a_ref, h_ref, o_ref):
    o_ref[...] = jnp.dot(a_ref[...], h_ref[...],
                         preferred_element_type=F32).astype(BF16)


def _masked_layernorm(y, dval, g, b):
    m = jnp.sum(y, axis=1, keepdims=True) / dval
    lane = jax.lax.broadcasted_iota(jnp.int32, y.shape, 1)
    d = jnp.where(lane < dval, y - m, 0.0)
    v = jnp.sum(d * d, axis=1, keepdims=True) / dval
    return d / jnp.sqrt(v + EPS) * g + b


def _spmm_ln_body(a_ref, h_ref, w_ref, b_ref, g_ref, bb_ref, o_ref, *, dval):
    z = jnp.dot(a_ref[...], h_ref[...], preferred_element_type=F32)
    y = jnp.dot(z.astype(BF16), w_ref[...], preferred_element_type=F32) + b_ref[...]
    t = jnp.tanh(_masked_layernorm(y, dval, g_ref[...], bb_ref[...]))
    o_ref[...] = t.astype(BF16)


def _spmm_final_body(a_ref, h_ref, w_ref, b_ref, g_ref, bb_ref, wo_ref, bo_ref,
                     o_ref, *, dval):
    z = jnp.dot(a_ref[...], h_ref[...], preferred_element_type=F32)
    y = jnp.dot(z.astype(BF16), w_ref[...], preferred_element_type=F32) + b_ref[...]
    t = jnp.tanh(_masked_layernorm(y, dval, g_ref[...], bb_ref[...]))
    o_ref[...] = (jnp.dot(t.astype(BF16), wo_ref[...], preferred_element_type=F32)
                  + bo_ref[...])


def _spmm(a_bf, h, body, extras, out_w, out_dtype, r_blk, interpret=False):
    n_rows, n_cols = a_bf.shape
    nblk = -(-n_rows // r_blk)
    f_in = h.shape[1]
    row_spec = pl.BlockSpec((r_blk, n_cols), lambda i: (i, 0))
    whole = lambda shape: pl.BlockSpec(shape, lambda i: (0, 0))
    in_specs = [row_spec, whole((n_cols, f_in))] + [whole(e.shape) for e in extras]
    out_spec = pl.BlockSpec((r_blk, out_w), lambda i: (i, 0))
    return pl.pallas_call(
        body,
        grid=(nblk,),
        in_specs=in_specs,
        out_specs=out_spec,
        out_shape=jax.ShapeDtypeStruct((nblk * r_blk, out_w), out_dtype),
        compiler_params=pltpu.CompilerParams(
            dimension_semantics=("arbitrary",)),
        interpret=interpret,
    )(a_bf, h, *extras)


def _cast_spmm(adj, h, r_blk, interpret=False):
    n_rows, n_cols = adj.shape
    f_in = h.shape[1]
    row_spec = pl.BlockSpec((r_blk, n_cols), lambda i: (i, 0))
    return pl.pallas_call(
        _cast_spmm_body,
        grid=(n_rows // r_blk,),
        in_specs=[row_spec, pl.BlockSpec((n_cols, f_in), lambda i: (0, 0))],
        out_specs=[row_spec, pl.BlockSpec((r_blk, f_in), lambda i: (i, 0))],
        out_shape=[jax.ShapeDtypeStruct((n_rows, n_cols), BF16),
                   jax.ShapeDtypeStruct((n_rows, f_in), BF16)],
        compiler_params=pltpu.CompilerParams(
            dimension_semantics=("arbitrary",)),
        interpret=interpret,
    )(adj, h)


def _mega_body(a_ref, h1_ref, b1_ref, g1_ref, bb1_ref,
               w2_ref, b2_ref, g2_ref, bb2_ref, wo_ref, bo_ref,
               o_ref, sa, sb, *, r, n, d1, d2, f2, nblk):
    p = pl.program_id(0)
    i = pl.program_id(1)
    i_eff = jnp.where(p % 2 == 1, nblk - 1 - i, i)
    row0 = i_eff * r

    @pl.when(p == 0)
    def _():
        z = jnp.dot(a_ref[...], h1_ref[...], preferred_element_type=F32)
        sa[pl.ds(row0, r), :] = z.astype(BF16)

    @pl.when(p == 1)
    def _():
        z = jnp.dot(a_ref[...], sa[:n, :], preferred_element_type=F32)
        sb[pl.ds(row0, r), :] = z.astype(BF16)

    @pl.when(p == 2)
    def _():
        z = jnp.dot(a_ref[...], sb[:n, :], preferred_element_type=F32)
        y = z + b1_ref[...]
        t = jnp.tanh(_masked_layernorm(y, d1, g1_ref[...], bb1_ref[...]))
        g = jnp.dot(t.astype(BF16), w2_ref[...], preferred_element_type=F32)
        sa[pl.ds(row0, r), :f2] = g.astype(BF16)

    @pl.when(p == 3)
    def _():
        z = jnp.dot(a_ref[...], sa[:n, :f2], preferred_element_type=F32)
        sb[pl.ds(row0, r), :f2] = z.astype(BF16)

    @pl.when(p == 4)
    def _():
        z = jnp.dot(a_ref[...], sb[:n, :f2], preferred_element_type=F32)
        sa[pl.ds(row0, r), :f2] = z.astype(BF16)

    @pl.when(p == 5)
    def _():
        z = jnp.dot(a_ref[...], sa[:n, :f2], preferred_element_type=F32)
        sb[pl.ds(row0, r), :f2] = z.astype(BF16)

    @pl.when(p == 6)
    def _():
        z = jnp.dot(a_ref[...], sb[:n, :f2], preferred_element_type=F32)
        y = z + b2_ref[...]
        t = jnp.tanh(_masked_layernorm(y, d2, g2_ref[...], bb2_ref[...]))
        o_ref[...] = (jnp.dot(t.astype(BF16), wo_ref[...],
                              preferred_element_type=F32) + bo_ref[...])


def _mega(a_bf, h1, extras, d1, d2, f2, d_out, r_blk, interpret=False):
    n, n_cols = a_bf.shape
    nblk = -(-n // r_blk)
    rows_pad = nblk * r_blk
    f_in = h1.shape[1]
    whole = lambda shape: pl.BlockSpec(shape, lambda p, i: (0, 0))
    serp = lambda p, i: (jnp.where(p % 2 == 1, nblk - 1 - i, i), 0)
    body = functools.partial(_mega_body, r=r_blk, n=n_cols, d1=d1, d2=d2, f2=f2,
                             nblk=nblk)
    return pl.pallas_call(
        body,
        grid=(7, nblk),
        in_specs=[pl.BlockSpec((r_blk, n_cols), serp),
                  whole((n_cols, f_in))] + [whole(e.shape) for e in extras],
        out_specs=pl.BlockSpec((r_blk, d_out),
                               lambda p, i: (jnp.where(p == 6, i, 0), 0)),
        out_shape=jax.ShapeDtypeStruct((rows_pad, d_out), F32),
        scratch_shapes=[pltpu.VMEM((rows_pad, f_in), BF16),
                        pltpu.VMEM((rows_pad, f_in), BF16)],
        compiler_params=pltpu.CompilerParams(
            dimension_semantics=("arbitrary", "arbitrary"),
            vmem_limit_bytes=64 * 1024 * 1024),
        interpret=interpret,
    )(a_bf, h1, *extras)


def _pad2(a, rows, cols):
    return jnp.pad(a, ((0, rows - a.shape[0]), (0, cols - a.shape[1])))


def _row(v, width=F):
    return jnp.pad(v, (0, width - v.shape[0])).reshape(1, width)


def _pipeline(x, adj, bn_g, bn_b, Wi, bi, W1, b1, ln1_g, ln1_b, W2, b2,
              ln2_g, ln2_b, Wo, bo, r_blk, r_big, interpret=False):
    d_out = Wo.shape[0]
    f2 = 128

    wi_t = _pad2(Wi.T.astype(BF16), Wi.shape[1], F)
    w1_t = _pad2(W1.T.astype(BF16), F, f2)
    w2_t = _pad2(W2.T.astype(BF16), f2, f2)
    wo_t = _pad2(Wo.T.astype(BF16), f2, d_out)

    h = pl.pallas_call(
        _prologue_body,
        out_shape=jax.ShapeDtypeStruct((x.shape[0], f2), BF16),
        interpret=interpret,
    )(x, bn_g.reshape(1, -1), bn_b.reshape(1, -1), wi_t, _row(bi), w1_t)

    a_bf, h = _cast_spmm(adj, h, r_blk, interpret)
    out = _mega(a_bf, h,
                [_row(b1, f2), _row(ln1_g, f2), _row(ln1_b, f2),
                 w2_t, _row(b2, f2), _row(ln2_g, f2), _row(ln2_b, f2),
                 wo_t, _row(bo, d_out)],
                W1.shape[0], W2.shape[0], f2, d_out, r_big, interpret)
    return out[:adj.shape[0]]


def kernel(x, adj, bn_g, bn_b, Wi, bi, W1, b1, ln1_g, ln1_b, W2, b2,
           ln2_g, ln2_b, Wo, bo):
    return _pipeline(x, adj, bn_g, bn_b, Wi, bi, W1, b1, ln1_g, ln1_b,
                     W2, b2, ln2_g, ln2_b, Wo, bo, r_blk=400, r_big=1024)


# speedup vs baseline: 1.0364x; 1.0066x over previous
import functools

import jax
import jax.numpy as jnp
from jax.experimental import pallas as pl
from jax.experimental.pallas import tpu as pltpu

F32 = jnp.float32
BF16 = jnp.bfloat16
F = 256
EPS = 1e-5


def _prologue_body(x_ref, g_ref, b_ref, w_ref, bi_ref, w1_ref, o_ref):
    x = x_ref[...]
    m = jnp.mean(x, axis=0, keepdims=True)
    v = jnp.mean((x - m) ** 2, axis=0, keepdims=True)
    xn = (x - m) / jnp.sqrt(v + EPS) * g_ref[...] + b_ref[...]
    h = jnp.dot(xn.astype(BF16), w_ref[...], preferred_element_type=F32)
    t = jnp.tanh(h + bi_ref[...]).astype(BF16)
    o_ref[...] = jnp.dot(t, w1_ref[...], preferred_element_type=F32).astype(BF16)


def _cast_spmm_body(a_ref, x_ref, g_ref, b_ref, w_ref, bi_ref, w1_ref,
                    abf_ref, o_ref, h0):
    @pl.when(pl.program_id(0) == 0)
    def _():
        x = x_ref[...]
        m = jnp.mean(x, axis=0, keepdims=True)
        v = jnp.mean((x - m) ** 2, axis=0, keepdims=True)
        xn = (x - m) / jnp.sqrt(v + EPS) * g_ref[...] + b_ref[...]
        h = jnp.dot(xn.astype(BF16), w_ref[...], preferred_element_type=F32)
        t = jnp.tanh(h + bi_ref[...]).astype(BF16)
        h0[...] = jnp.dot(t, w1_ref[...], preferred_element_type=F32).astype(BF16)

    abf = a_ref[...].astype(BF16)
    abf_ref[...] = abf
    o_ref[...] = jnp.dot(abf, h0[...],
                         preferred_element_type=F32).astype(BF16)


def _spmm_plain_body(a_ref, h_ref, o_ref):
    o_ref[...] = jnp.dot(a_ref[...], h_ref[...],
                         preferred_element_type=F32).astype(BF16)


def _masked_layernorm(y, dval, g, b):
    m = jnp.sum(y, axis=1, keepdims=True) / dval
    lane = jax.lax.broadcasted_iota(jnp.int32, y.shape, 1)
    d = jnp.where(lane < dval, y - m, 0.0)
    v = jnp.sum(d * d, axis=1, keepdims=True) / dval
    return d / jnp.sqrt(v + EPS) * g + b


def _spmm_ln_body(a_ref, h_ref, w_ref, b_ref, g_ref, bb_ref, o_ref, *, dval):
    z = jnp.dot(a_ref[...], h_ref[...], preferred_element_type=F32)
    y = jnp.dot(z.astype(BF16), w_ref[...], preferred_element_type=F32) + b_ref[...]
    t = jnp.tanh(_masked_layernorm(y, dval, g_ref[...], bb_ref[...]))
    o_ref[...] = t.astype(BF16)


def _spmm_final_body(a_ref, h_ref, w_ref, b_ref, g_ref, bb_ref, wo_ref, bo_ref,
                     o_ref, *, dval):
    z = jnp.dot(a_ref[...], h_ref[...], preferred_element_type=F32)
    y = jnp.dot(z.astype(BF16), w_ref[...], preferred_element_type=F32) + b_ref[...]
    t = jnp.tanh(_masked_layernorm(y, dval, g_ref[...], bb_ref[...]))
    o_ref[...] = (jnp.dot(t.astype(BF16), wo_ref[...], preferred_element_type=F32)
                  + bo_ref[...])


def _spmm(a_bf, h, body, extras, out_w, out_dtype, r_blk, interpret=False):
    n_rows, n_cols = a_bf.shape
    nblk = -(-n_rows // r_blk)
    f_in = h.shape[1]
    row_spec = pl.BlockSpec((r_blk, n_cols), lambda i: (i, 0))
    whole = lambda shape: pl.BlockSpec(shape, lambda i: (0, 0))
    in_specs = [row_spec, whole((n_cols, f_in))] + [whole(e.shape) for e in extras]
    out_spec = pl.BlockSpec((r_blk, out_w), lambda i: (i, 0))
    return pl.pallas_call(
        body,
        grid=(nblk,),
        in_specs=in_specs,
        out_specs=out_spec,
        out_shape=jax.ShapeDtypeStruct((nblk * r_blk, out_w), out_dtype),
        compiler_params=pltpu.CompilerParams(
            dimension_semantics=("arbitrary",)),
        interpret=interpret,
    )(a_bf, h, *extras)


def _cast_spmm(adj, x, bn_g, bn_b, wi_t, bi_r, w1_t, f_in, r_blk,
               interpret=False):
    n_rows, n_cols = adj.shape
    row_spec = pl.BlockSpec((r_blk, n_cols), lambda i: (i, 0))
    whole = lambda shape: pl.BlockSpec(shape, lambda i: (0, 0))
    return pl.pallas_call(
        _cast_spmm_body,
        grid=(n_rows // r_blk,),
        in_specs=[row_spec, whole(x.shape), whole(bn_g.shape), whole(bn_b.shape),
                  whole(wi_t.shape), whole(bi_r.shape), whole(w1_t.shape)],
        out_specs=[row_spec, pl.BlockSpec((r_blk, f_in), lambda i: (i, 0))],
        out_shape=[jax.ShapeDtypeStruct((n_rows, n_cols), BF16),
                   jax.ShapeDtypeStruct((n_rows, f_in), BF16)],
        scratch_shapes=[pltpu.VMEM((n_rows, f_in), BF16)],
        compiler_params=pltpu.CompilerParams(
            dimension_semantics=("arbitrary",),
            vmem_limit_bytes=64 * 1024 * 1024),
        interpret=interpret,
    )(adj, x, bn_g, bn_b, wi_t, bi_r, w1_t)


def _mega_body(a_ref, h1_ref, b1_ref, g1_ref, bb1_ref,
               w2_ref, b2_ref, g2_ref, bb2_ref, wo_ref, bo_ref,
               o_ref, sa, sb, *, r, n, d1, d2, f2, nblk):
    p = pl.program_id(0)
    i = pl.program_id(1)
    i_eff = jnp.where(p % 2 == 1, nblk - 1 - i, i)
    row0 = i_eff * r

    @pl.when(p == 0)
    def _():
        z = jnp.dot(a_ref[...], h1_ref[...], preferred_element_type=F32)
        sa[pl.ds(row0, r), :] = z.astype(BF16)

    @pl.when(p == 1)
    def _():
        z = jnp.dot(a_ref[...], sa[:n, :], preferred_element_type=F32)
        sb[pl.ds(row0, r), :] = z.astype(BF16)

    @pl.when(p == 2)
    def _():
        z = jnp.dot(a_ref[...], sb[:n, :], preferred_element_type=F32)
        y = z + b1_ref[...]
        t = jnp.tanh(_masked_layernorm(y, d1, g1_ref[...], bb1_ref[...]))
        g = jnp.dot(t.astype(BF16), w2_ref[...], preferred_element_type=F32)
        sa[pl.ds(row0, r), :f2] = g.astype(BF16)

    @pl.when(p == 3)
    def _():
        z = jnp.dot(a_ref[...], sa[:n, :f2], preferred_element_type=F32)
        sb[pl.ds(row0, r), :f2] = z.astype(BF16)

    @pl.when(p == 4)
    def _():
        z = jnp.dot(a_ref[...], sb[:n, :f2], preferred_element_type=F32)
        sa[pl.ds(row0, r), :f2] = z.astype(BF16)

    @pl.when(p == 5)
    def _():
        z = jnp.dot(a_ref[...], sa[:n, :f2], preferred_element_type=F32)
        sb[pl.ds(row0, r), :f2] = z.astype(BF16)

    @pl.when(p == 6)
    def _():
        z = jnp.dot(a_ref[...], sb[:n, :f2], preferred_element_type=F32)
        y = z + b2_ref[...]
        t = jnp.tanh(_masked_layernorm(y, d2, g2_ref[...], bb2_ref[...]))
        o_ref[...] = (jnp.dot(t.astype(BF16), wo_ref[...],
                              preferred_element_type=F32) + bo_ref[...])


def _mega(a_bf, h1, extras, d1, d2, f2, d_out, r_blk, interpret=False):
    n, n_cols = a_bf.shape
    nblk = -(-n // r_blk)
    rows_pad = nblk * r_blk
    f_in = h1.shape[1]
    whole = lambda shape: pl.BlockSpec(shape, lambda p, i: (0, 0))
    serp = lambda p, i: (jnp.where(p % 2 == 1, nblk - 1 - i, i), 0)
    body = functools.partial(_mega_body, r=r_blk, n=n_cols, d1=d1, d2=d2, f2=f2,
                             nblk=nblk)
    return pl.pallas_call(
        body,
        grid=(7, nblk),
        in_specs=[pl.BlockSpec((r_blk, n_cols), serp),
                  whole((n_cols, f_in))] + [whole(e.shape) for e in extras],
        out_specs=pl.BlockSpec((r_blk, d_out),
                               lambda p, i: (jnp.where(p == 6, i, 0), 0)),
        out_shape=jax.ShapeDtypeStruct((rows_pad, d_out), F32),
        scratch_shapes=[pltpu.VMEM((rows_pad, f_in), BF16),
                        pltpu.VMEM((rows_pad, f_in), BF16)],
        compiler_params=pltpu.CompilerParams(
            dimension_semantics=("arbitrary", "arbitrary"),
            vmem_limit_bytes=64 * 1024 * 1024),
        interpret=interpret,
    )(a_bf, h1, *extras)


def _pad2(a, rows, cols):
    return jnp.pad(a, ((0, rows - a.shape[0]), (0, cols - a.shape[1])))


def _row(v, width=F):
    return jnp.pad(v, (0, width - v.shape[0])).reshape(1, width)


def _pipeline(x, adj, bn_g, bn_b, Wi, bi, W1, b1, ln1_g, ln1_b, W2, b2,
              ln2_g, ln2_b, Wo, bo, r_blk, r_big, interpret=False):
    d_out = Wo.shape[0]
    f2 = 128

    wi_t = _pad2(Wi.T.astype(BF16), Wi.shape[1], F)
    w1_t = _pad2(W1.T.astype(BF16), F, f2)
    w2_t = _pad2(W2.T.astype(BF16), f2, f2)
    wo_t = _pad2(Wo.T.astype(BF16), f2, d_out)

    a_bf, h = _cast_spmm(adj, x, bn_g.reshape(1, -1), bn_b.reshape(1, -1),
                         wi_t, _row(bi), w1_t, f2, r_blk, interpret)
    out = _mega(a_bf, h,
                [_row(b1, f2), _row(ln1_g, f2), _row(ln1_b, f2),
                 w2_t, _row(b2, f2), _row(ln2_g, f2), _row(ln2_b, f2),
                 wo_t, _row(bo, d_out)],
                W1.shape[0], W2.shape[0], f2, d_out, r_big, interpret)
    return out[:adj.shape[0]]


def kernel(x, adj, bn_g, bn_b, Wi, bi, W1, b1, ln1_g, ln1_b, W2, b2,
           ln2_g, ln2_b, Wo, bo):
    return _pipeline(x, adj, bn_g, bn_b, Wi, bi, W1, b1, ln1_g, ln1_b,
                     W2, b2, ln2_g, ln2_b, Wo, bo, r_blk=400, r_big=1024)


# speedup vs baseline: 1.0561x; 1.0190x over previous
import functools

import jax
import jax.numpy as jnp
from jax.experimental import pallas as pl
from jax.experimental.pallas import tpu as pltpu

F32 = jnp.float32
BF16 = jnp.bfloat16
F = 256
EPS = 1e-5


def _masked_layernorm(y, dval, g, b):
    m = jnp.sum(y, axis=1, keepdims=True) / dval
    lane = jax.lax.broadcasted_iota(jnp.int32, y.shape, 1)
    d = jnp.where(lane < dval, y - m, 0.0)
    v = jnp.sum(d * d, axis=1, keepdims=True) / dval
    return d / jnp.sqrt(v + EPS) * g + b


def _cast_spmm_body(a_ref, x_ref, g_ref, b_ref, w_ref, bi_ref, w1_ref,
                    abf_ref, o_ref, h0):
    @pl.when(pl.program_id(0) == 0)
    def _():
        x = x_ref[...]
        m = jnp.mean(x, axis=0, keepdims=True)
        v = jnp.mean((x - m) ** 2, axis=0, keepdims=True)
        xn = (x - m) / jnp.sqrt(v + EPS) * g_ref[...] + b_ref[...]
        h = jnp.dot(xn.astype(BF16), w_ref[...], preferred_element_type=F32)
        t = jnp.tanh(h + bi_ref[...]).astype(BF16)
        h0[...] = jnp.dot(t, w1_ref[...], preferred_element_type=F32).astype(BF16)

    abf = a_ref[...].astype(BF16)
    abf_ref[...] = abf
    o_ref[...] = jnp.dot(abf, h0[...],
                         preferred_element_type=F32).astype(BF16)


def _cast_spmm(adj, x, bn_g, bn_b, wi_t, bi_r, w1_t, f_in, r_blk,
               interpret=False):
    n_rows, n_cols = adj.shape
    row_spec = pl.BlockSpec((r_blk, n_cols), lambda i: (i, 0))
    whole = lambda shape: pl.BlockSpec(shape, lambda i: (0, 0))
    return pl.pallas_call(
        _cast_spmm_body,
        grid=(n_rows // r_blk,),
        in_specs=[row_spec, whole(x.shape), whole(bn_g.shape), whole(bn_b.shape),
                  whole(wi_t.shape), whole(bi_r.shape), whole(w1_t.shape)],
        out_specs=[row_spec, pl.BlockSpec((r_blk, f_in), lambda i: (i, 0))],
        out_shape=[jax.ShapeDtypeStruct((n_rows, n_cols), BF16),
                   jax.ShapeDtypeStruct((n_rows, f_in), BF16)],
        scratch_shapes=[pltpu.VMEM((n_rows, f_in), BF16)],
        compiler_params=pltpu.CompilerParams(
            dimension_semantics=("arbitrary",),
            vmem_limit_bytes=64 * 1024 * 1024),
        interpret=interpret,
    )(adj, x, bn_g, bn_b, wi_t, bi_r, w1_t)


def _mega_body(a_ref, h1_ref, b1_ref, g1_ref, bb1_ref,
               w2_ref, b2_ref, g2_ref, bb2_ref, wo_ref, bo_ref,
               o_ref, sa, sb, *, r, n, d1, d2, f2, nblk):
    p = pl.program_id(0)
    i = pl.program_id(1)
    i_eff = jnp.where(p % 2 == 1, nblk - 1 - i, i)
    row0 = i_eff * r

    @pl.when(p == 0)
    def _():
        z = jnp.dot(a_ref[...], h1_ref[...], preferred_element_type=F32)
        sa[pl.ds(row0, r), :] = z.astype(BF16)

    @pl.when(p == 1)
    def _():
        z = jnp.dot(a_ref[...], sa[:n, :], preferred_element_type=F32)
        sb[pl.ds(row0, r), :] = z.astype(BF16)

    @pl.when(p == 2)
    def _():
        z = jnp.dot(a_ref[...], sb[:n, :], preferred_element_type=F32)
        y = z + b1_ref[...]
        t = jnp.tanh(_masked_layernorm(y, d1, g1_ref[...], bb1_ref[...]))
        g = jnp.dot(t.astype(BF16), w2_ref[...], preferred_element_type=F32)
        sa[pl.ds(row0, r), :f2] = g.astype(BF16)

    @pl.when(p == 3)
    def _():
        z = jnp.dot(a_ref[...], sa[:n, :f2], preferred_element_type=F32)
        sb[pl.ds(row0, r), :f2] = z.astype(BF16)

    @pl.when(p == 4)
    def _():
        z = jnp.dot(a_ref[...], sb[:n, :f2], preferred_element_type=F32)
        sa[pl.ds(row0, r), :f2] = z.astype(BF16)

    @pl.when(p == 5)
    def _():
        z = jnp.dot(a_ref[...], sa[:n, :f2], preferred_element_type=F32)
        sb[pl.ds(row0, r), :f2] = z.astype(BF16)

    @pl.when(p == 6)
    def _():
        z = jnp.dot(a_ref[...], sb[:n, :f2], preferred_element_type=F32)
        y = z + b2_ref[...]
        t = jnp.tanh(_masked_layernorm(y, d2, g2_ref[...], bb2_ref[...]))
        o_ref[...] = (jnp.dot(t.astype(BF16), wo_ref[...],
                              preferred_element_type=F32) + bo_ref[...])


def _mega(a_bf, h1, extras, d1, d2, f2, d_out, r_blk, interpret=False):
    n, n_cols = a_bf.shape
    nblk = -(-n // r_blk)
    rows_pad = nblk * r_blk
    f_in = h1.shape[1]
    whole = lambda shape: pl.BlockSpec(shape, lambda p, i: (0, 0))
    serp = lambda p, i: (jnp.where(p % 2 == 1, nblk - 1 - i, i), 0)
    body = functools.partial(_mega_body, r=r_blk, n=n_cols, d1=d1, d2=d2, f2=f2,
                             nblk=nblk)
    return pl.pallas_call(
        body,
        grid=(7, nblk),
        in_specs=[pl.BlockSpec((r_blk, n_cols), serp),
                  whole((n_cols, f_in))] + [whole(e.shape) for e in extras],
        out_specs=pl.BlockSpec((r_blk, d_out),
                               lambda p, i: (jnp.where(p == 6, i, 0), 0)),
        out_shape=jax.ShapeDtypeStruct((rows_pad, d_out), F32),
        scratch_shapes=[pltpu.VMEM((rows_pad, f_in), BF16),
                        pltpu.VMEM((rows_pad, f_in), BF16)],
        compiler_params=pltpu.CompilerParams(
            dimension_semantics=("arbitrary", "arbitrary"),
            vmem_limit_bytes=64 * 1024 * 1024),
        interpret=interpret,
    )(a_bf, h1, *extras)


def _pad2(a, rows, cols):
    return jnp.pad(a, ((0, rows - a.shape[0]), (0, cols - a.shape[1])))


def _row(v, width=F):
    return jnp.pad(v, (0, width - v.shape[0])).reshape(1, width)


def _pipeline(x, adj, bn_g, bn_b, Wi, bi, W1, b1, ln1_g, ln1_b, W2, b2,
              ln2_g, ln2_b, Wo, bo, r_blk, r_big, interpret=False):
    d_out = Wo.shape[0]
    f2 = 128

    wi_t = _pad2(Wi.T.astype(BF16), Wi.shape[1], F)
    w1_t = _pad2(W1.T.astype(BF16), F, f2)
    w2_t = _pad2(W2.T.astype(BF16), f2, f2)
    wo_t = _pad2(Wo.T.astype(BF16), f2, d_out)

    a_bf, h = _cast_spmm(adj, x, bn_g.reshape(1, -1), bn_b.reshape(1, -1),
                         wi_t, _row(bi), w1_t, f2, r_blk, interpret)
    out = _mega(a_bf, h,
                [_row(b1, f2), _row(ln1_g, f2), _row(ln1_b, f2),
                 w2_t, _row(b2, f2), _row(ln2_g, f2), _row(ln2_b, f2),
                 wo_t, _row(bo, d_out)],
                W1.shape[0], W2.shape[0], f2, d_out, r_big, interpret)
    return out[:adj.shape[0]]


def kernel(x, adj, bn_g, bn_b, Wi, bi, W1, b1, ln1_g, ln1_b, W2, b2,
           ln2_g, ln2_b, Wo, bo):
    return _pipeline(x, adj, bn_g, bn_b, Wi, bi, W1, b1, ln1_g, ln1_b,
                     W2, b2, ln2_g, ln2_b, Wo, bo, r_blk=400, r_big=1024)
